# baseline (speedup 1.0000x reference)
import functools

import jax
import jax.numpy as jnp
from jax import lax
from jax.experimental import pallas as pl
from jax.experimental.pallas import tpu as pltpu
from jax.experimental.pallas import tpu_sc as plsc

N = 10000
E = 160000
D_IN = 258
D = 256

NC = 2
NS = 16
NW = NC * NS

EPW = 5120
E_PAD = NW * EPW
NPAD = 10240
NT = NPAD // NW
DUMPLOC = NT
ACC_ROWS = NT + 8
SCH = 512
NCH = E_PAD // SCH
RB = 128


def _mm_body(x_ref, w_ref, asd_ref, we_ref, ae_ref, h_ref, sd_ref):
    h = jnp.dot(x_ref[...], w_ref[...], preferred_element_type=jnp.float32)
    h_ref[...] = h
    sd = jnp.dot(h, asd_ref[...], preferred_element_type=jnp.float32)
    c = jnp.sum(we_ref[...] * ae_ref[...])
    col = lax.broadcasted_iota(jnp.int32, sd.shape, 1)
    sd_ref[...] = sd + jnp.where(col == 2, c, 0.0)


def _project(x, W, asd8, We, ae2):
    bm = 1000
    return pl.pallas_call(
        _mm_body,
        grid=(N // bm,),
        in_specs=[
            pl.BlockSpec((bm, D_IN), lambda i: (i, 0)),
            pl.BlockSpec((D_IN, D), lambda i: (0, 0)),
            pl.BlockSpec((D, 8), lambda i: (0, 0)),
            pl.BlockSpec((1, D), lambda i: (0, 0)),
            pl.BlockSpec((1, D), lambda i: (0, 0)),
        ],
        out_specs=[
            pl.BlockSpec((bm, D), lambda i: (i, 0)),
            pl.BlockSpec((bm, 8), lambda i: (i, 0)),
        ],
        out_shape=[
            jax.ShapeDtypeStruct((N, D), jnp.float32),
            jax.ShapeDtypeStruct((N, 8), jnp.float32),
        ],
    )(x, W, asd8, We, ae2)


_sc_mesh = plsc.VectorSubcoreMesh(core_axis_name="c", subcore_axis_name="s")


@functools.partial(
    pl.kernel,
    out_type=(
        jax.ShapeDtypeStruct((E_PAD,), jnp.float32),
        jax.ShapeDtypeStruct((NW * NPAD,), jnp.float32),
    ),
    mesh=_sc_mesh,
    compiler_params=pltpu.CompilerParams(needs_layout_passes=False),
    scratch_types=[
        pltpu.VMEM((N,), jnp.float32),
        pltpu.VMEM((N,), jnp.float32),
        pltpu.VMEM((16,), jnp.float32),
        pltpu.VMEM((EPW,), jnp.int32),
        pltpu.VMEM((EPW,), jnp.int32),
        pltpu.VMEM((EPW,), jnp.float32),
        pltpu.VMEM((EPW,), jnp.float32),
        pltpu.VMEM((NPAD,), jnp.float32),
    ],
)
def _edge_logits(src_hbm, dst_hbm, ea_hbm, s_hbm, d_hbm, c_hbm,
                 ex_hbm, dp_hbm,
                 s_v, d_v, c_v, src_v, dst_v, ea_v, ex_v, den_v):
    wid = lax.axis_index("s") * NC + lax.axis_index("c")
    base = wid * EPW

    def zero(i, _):
        den_v[pl.ds(i * 16, 16)] = jnp.zeros((16,), jnp.float32)
        return 0
    lax.fori_loop(0, NPAD // 16, zero, 0)

    pltpu.sync_copy(s_hbm, s_v)
    pltpu.sync_copy(d_hbm, d_v)
    pltpu.sync_copy(c_hbm, c_v)
    pltpu.sync_copy(src_hbm.at[pl.ds(base, EPW)], src_v)
    pltpu.sync_copy(dst_hbm.at[pl.ds(base, EPW)], dst_v)
    pltpu.sync_copy(ea_hbm.at[pl.ds(base, EPW)], ea_v)
    cc = c_v[...]

    def body(i, _):
        sl = pl.ds(i * 16, 16)
        sv = src_v[sl]
        dv = dst_v[sl]
        sg = plsc.load_gather(s_v, [sv])
        dg = plsc.load_gather(d_v, [dv])
        logit = sg + dg + cc * ea_v[sl]
        logit = jnp.maximum(logit, 0.2 * logit)
        ex = jnp.exp(logit)
        ex_v[sl] = ex
        eids = base + i * 16 + lax.iota(jnp.int32, 16)
        plsc.addupdate_scatter(den_v, [dv], ex, mask=eids < E)
        return 0
    lax.fori_loop(0, EPW // 16, body, 0)

    pltpu.sync_copy(ex_v, ex_hbm.at[pl.ds(base, EPW)])
    pltpu.sync_copy(den_v, dp_hbm.at[pl.ds(wid * NPAD, NPAD)])


@functools.partial(
    pl.kernel,
    out_type=(
        jax.ShapeDtypeStruct((NPAD, D), jnp.float32),
        jax.ShapeDtypeStruct((E_PAD,), jnp.float32),
    ),
    mesh=_sc_mesh,
    compiler_params=pltpu.CompilerParams(needs_layout_passes=False),
    scratch_types=[
        pltpu.VMEM((ACC_ROWS * D,), jnp.float32),
        pltpu.VMEM((NT + 16,), jnp.float32),
        pltpu.VMEM((NT + 16,), jnp.float32),
        pltpu.VMEM((NT,), jnp.int32),
        pltpu.VMEM((NT,), jnp.float32),
        pltpu.VMEM((SCH,), jnp.int32),
        pltpu.VMEM((SCH + 16,), jnp.int32),
        pltpu.VMEM((SCH + 16,), jnp.int32),
        pltpu.VMEM((RB,), jnp.int32),
        pltpu.VMEM((RB,), jnp.int32),
        pltpu.VMEM((RB,), jnp.int32),
        pltpu.VMEM((RB,), jnp.float32),
        pltpu.VMEM((RB,), jnp.float32),
        pltpu.VMEM((RB,), jnp.float32),
        pltpu.VMEM((RB, D), jnp.float32),
        pltpu.VMEM((D,), jnp.float32),
        pltpu.VMEM((D,), jnp.float32),
    ],
)
def _aggregate(src_hbm, dst_hbm, ea_hbm, ex_hbm, dp_hbm, h_hbm, w0_hbm, b_hbm,
               outp_hbm, alpha_hbm,
               accf_v, den_v, beta_v, dpidx_b, dpsl_b,
               scan_b, posc_b, dstc_b,
               pos_b, dloc_b, srcg_b, exg_b, eag_b, alphab_b,
               rows_v, w0_v, b_v):
    wid = lax.axis_index("s") * NC + lax.axis_index("c")
    base_node = wid * NT
    iota16 = lax.iota(jnp.int32, 16)
    zeros16 = jnp.zeros((16,), jnp.float32)

    def zacc(i, _):
        for u in range(4):
            accf_v[pl.ds(i * 64 + u * 16, 16)] = zeros16
        return 0
    lax.fori_loop(0, ACC_ROWS * D // 64, zacc, 0)

    def zsmall(i, _):
        den_v[pl.ds(i * 16, 16)] = zeros16
        beta_v[pl.ds(i * 16, 16)] = zeros16
        return 0
    lax.fori_loop(0, (NT + 16) // 16, zsmall, 0)

    def zidx(i, _):
        dpidx_b[pl.ds(i * 16, 16)] = base_node + i * 16 + iota16
        return 0
    lax.fori_loop(0, NT // 16, zidx, 0)

    pltpu.sync_copy(w0_hbm, w0_v)
    pltpu.sync_copy(b_hbm, b_v)

    for p in range(NW):
        pltpu.sync_copy(dp_hbm.at[dpidx_b], dpsl_b)

        def dred(i, _):
            sl = pl.ds(i * 16, 16)
            den_v[sl] = den_v[sl] + dpsl_b[sl]
            if p < NW - 1:
                dpidx_b[sl] = dpidx_b[sl] + NPAD
            return 0
        lax.fori_loop(0, NT // 16, dred, 0)

    def chunk(ch, _):
        ebch = ch * SCH
        pltpu.sync_copy(dst_hbm.at[pl.ds(ebch, SCH)], scan_b)
        for i in range((SCH + 16) // 16):
            posc_b[pl.ds(i * 16, 16)] = jnp.full((16,), E, jnp.int32)
        cnt = jnp.int32(0)
        for jv in range(SCH // 16):
            sl = pl.ds(jv * 16, 16)
            dv = scan_b[sl]
            eidv = ebch + jv * 16 + iota16
            m = (dv >= base_node) & (dv < base_node + NT) & (eidv < E)
            plsc.store_compressed(posc_b.at[pl.ds(cnt, 16)], eidv, mask=m)
            plsc.store_compressed(dstc_b.at[pl.ds(cnt, 16)], dv, mask=m)
            cnt = cnt + jnp.sum(m.astype(jnp.int32))
        nbt = (cnt + RB - 1) // RB

        def batch(bt, _):
            off = bt * RB
            for k in range(RB // 16):
                koff = off + k * 16
                pv = posc_b[pl.ds(koff, 16)]
                dvv = dstc_b[pl.ds(koff, 16)]
                vmask = koff + iota16 < cnt
                pos_b[pl.ds(k * 16, 16)] = pv
                dloc_b[pl.ds(k * 16, 16)] = jnp.where(
                    vmask, dvv - base_node, DUMPLOC)
            pltpu.sync_copy(src_hbm.at[pos_b], srcg_b)
            pltpu.sync_copy(ex_hbm.at[pos_b], exg_b)
            pltpu.sync_copy(ea_hbm.at[pos_b], eag_b)
            pltpu.sync_copy(h_hbm.at[srcg_b], rows_v)
            for k in range(RB // 16):
                sl = pl.ds(k * 16, 16)
                dlocv = dloc_b[sl]
                denv = plsc.load_gather(den_v, [dlocv])
                alpha = exg_b[sl] / (denv + 1e-16)
                vmask = off + k * 16 + iota16 < cnt
                alpha = jnp.where(vmask, alpha, 0.0)
                alphab_b[sl] = alpha
                plsc.addupdate_scatter(
                    beta_v, [dlocv], alpha * eag_b[sl], mask=vmask)
            pltpu.sync_copy(alphab_b, alpha_hbm.at[pos_b])

            def rowacc(r, _):
                r16 = jnp.broadcast_to(r, (16,)).astype(jnp.int32)
                dloc16 = plsc.load_gather(dloc_b, [r16])
                a16 = plsc.load_gather(alphab_b, [r16])
                addr0 = dloc16 * D + iota16
                for q in range(D // 16):
                    plsc.addupdate_scatter(
                        accf_v, [addr0 + q * 16],
                        rows_v[r, pl.ds(q * 16, 16)] * a16)
                return 0
            lax.fori_loop(0, RB, rowacc, 0)
            return 0
        lax.fori_loop(0, nbt, batch, 0)
        return 0
    lax.fori_loop(0, NCH, chunk, 0)

    def fin(g, _):
        for j in range(16):
            row = g * 16 + j
            b16 = plsc.load_gather(
                beta_v, [jnp.broadcast_to(row, (16,)).astype(jnp.int32)])
            for q in range(D // 16):
                qs = pl.ds(q * 16, 16)
                rows_v[j, qs] = (accf_v[pl.ds(row * D + q * 16, 16)]
                                 + b16 * w0_v[qs] + b_v[qs])
        pltpu.sync_copy(rows_v.at[pl.ds(0, 16)],
                        outp_hbm.at[pl.ds(base_node + g * 16, 16)])
        return 0
    lax.fori_loop(0, NT // 16, fin, 0)


def kernel(x, edge_index, edge_attr, W, We, a_src, a_dst, a_edge, b):
    src = jnp.pad(edge_index[0], (0, E_PAD - E))
    dst = jnp.pad(edge_index[1], (0, E_PAD - E))
    ea = jnp.pad(edge_attr[:, 0], (0, E_PAD - E))
    asd8 = jnp.zeros((D, 8), jnp.float32).at[:, 0].set(a_src).at[:, 1].set(a_dst)
    ae2 = a_edge.reshape(1, D)

    h, sd = _project(x, W, asd8, We, ae2)
    s = sd[:, 0]
    d = sd[:, 1]
    cvec = sd[0:16, 2]

    ex, dparts = _edge_logits(src, dst, ea, s, d, cvec)
    outp, alpha = _aggregate(src, dst, ea, ex, dparts, h, We[0], b)

    return outp[:N], alpha[:E]


# speedup vs baseline: 146.9880x; 146.9880x over previous
import functools

import jax
import jax.numpy as jnp
from jax import lax
from jax.experimental import pallas as pl
from jax.experimental.pallas import tpu as pltpu
from jax.experimental.pallas import tpu_sc as plsc

N = 10000
E = 160000
D_IN = 258
D = 256

NC = 2
NS = 16
NW = NC * NS

EPW = 5120
E_PAD = NW * EPW
NPAD = 10240
NT = NPAD // NW
DUMPLOC = NT
ACC_ROWS = NT + 8
SCH = 2048
NCH = E_PAD // SCH
RB = 96
OWN_MAX = 6144


def _mm_body(x_ref, w_ref, asd_ref, we_ref, ae_ref, h_ref, sd_ref):
    h = jnp.dot(x_ref[...], w_ref[...], preferred_element_type=jnp.float32)
    h_ref[...] = h
    sd = jnp.dot(h, asd_ref[...], preferred_element_type=jnp.float32)
    c = jnp.sum(we_ref[...] * ae_ref[...])
    col = lax.broadcasted_iota(jnp.int32, sd.shape, 1)
    sd_ref[...] = sd + jnp.where(col == 2, c, 0.0)


def _project(x, W, asd8, We, ae2):
    bm = 1000
    return pl.pallas_call(
        _mm_body,
        grid=(N // bm,),
        in_specs=[
            pl.BlockSpec((bm, D_IN), lambda i: (i, 0)),
            pl.BlockSpec((D_IN, D), lambda i: (0, 0)),
            pl.BlockSpec((D, 8), lambda i: (0, 0)),
            pl.BlockSpec((1, D), lambda i: (0, 0)),
            pl.BlockSpec((1, D), lambda i: (0, 0)),
        ],
        out_specs=[
            pl.BlockSpec((bm, D), lambda i: (i, 0)),
            pl.BlockSpec((bm, 8), lambda i: (i, 0)),
        ],
        out_shape=[
            jax.ShapeDtypeStruct((N, D), jnp.float32),
            jax.ShapeDtypeStruct((N, 8), jnp.float32),
        ],
    )(x, W, asd8, We, ae2)


_sc_mesh = plsc.VectorSubcoreMesh(core_axis_name="c", subcore_axis_name="s")


@functools.partial(
    pl.kernel,
    out_type=(
        jax.ShapeDtypeStruct((E_PAD,), jnp.float32),
        jax.ShapeDtypeStruct((NW * NPAD,), jnp.float32),
    ),
    mesh=_sc_mesh,
    compiler_params=pltpu.CompilerParams(needs_layout_passes=False),
    scratch_types=[
        pltpu.VMEM((N,), jnp.float32),
        pltpu.VMEM((N,), jnp.float32),
        pltpu.VMEM((16,), jnp.float32),
        pltpu.VMEM((EPW,), jnp.int32),
        pltpu.VMEM((EPW,), jnp.int32),
        pltpu.VMEM((EPW,), jnp.float32),
        pltpu.VMEM((EPW,), jnp.float32),
        pltpu.VMEM((NPAD,), jnp.float32),
    ],
)
def _edge_logits(src_hbm, dst_hbm, ea_hbm, s_hbm, d_hbm, c_hbm,
                 ex_hbm, dp_hbm,
                 s_v, d_v, c_v, src_v, dst_v, ea_v, ex_v, den_v):
    wid = lax.axis_index("s") * NC + lax.axis_index("c")
    base = wid * EPW

    def zero(i, _):
        den_v[pl.ds(i * 16, 16)] = jnp.zeros((16,), jnp.float32)
        return 0
    lax.fori_loop(0, NPAD // 16, zero, 0)

    pltpu.sync_copy(s_hbm, s_v)
    pltpu.sync_copy(d_hbm, d_v)
    pltpu.sync_copy(c_hbm, c_v)
    pltpu.sync_copy(src_hbm.at[pl.ds(base, EPW)], src_v)
    pltpu.sync_copy(dst_hbm.at[pl.ds(base, EPW)], dst_v)
    pltpu.sync_copy(ea_hbm.at[pl.ds(base, EPW)], ea_v)
    cc = c_v[...]

    def body(i, _):
        sl = pl.ds(i * 16, 16)
        sv = src_v[sl]
        dv = dst_v[sl]
        sg = plsc.load_gather(s_v, [sv])
        dg = plsc.load_gather(d_v, [dv])
        logit = sg + dg + cc * ea_v[sl]
        logit = jnp.maximum(logit, 0.2 * logit)
        ex = jnp.exp(logit)
        ex_v[sl] = ex
        eids = base + i * 16 + lax.iota(jnp.int32, 16)
        plsc.addupdate_scatter(den_v, [dv], ex, mask=eids < E)
        return 0
    lax.fori_loop(0, EPW // 16, body, 0)

    pltpu.sync_copy(ex_v, ex_hbm.at[pl.ds(base, EPW)])
    pltpu.sync_copy(den_v, dp_hbm.at[pl.ds(wid * NPAD, NPAD)])


@functools.partial(
    pl.kernel,
    out_type=(
        jax.ShapeDtypeStruct((NPAD, D), jnp.float32),
        jax.ShapeDtypeStruct((E_PAD,), jnp.float32),
    ),
    mesh=_sc_mesh,
    compiler_params=pltpu.CompilerParams(needs_layout_passes=False),
    scratch_types=[
        pltpu.VMEM((ACC_ROWS * D,), jnp.float32),
        pltpu.VMEM((NT + 16,), jnp.float32),
        pltpu.VMEM((NT + 16,), jnp.float32),
        pltpu.VMEM((NT,), jnp.int32),
        pltpu.VMEM((NT,), jnp.float32),
        pltpu.VMEM((SCH,), jnp.int32),
        pltpu.VMEM((OWN_MAX + RB + 16,), jnp.int32),
        pltpu.VMEM((OWN_MAX + RB + 16,), jnp.int32),
        pltpu.VMEM((RB,), jnp.int32),
        pltpu.VMEM((RB,), jnp.int32),
        pltpu.VMEM((RB,), jnp.float32),
        pltpu.VMEM((RB,), jnp.float32),
        pltpu.VMEM((RB,), jnp.float32),
        pltpu.VMEM((RB, D), jnp.float32),
        pltpu.VMEM((D,), jnp.float32),
        pltpu.VMEM((D,), jnp.float32),
    ],
)
def _aggregate(src_hbm, dst_hbm, ea_hbm, ex_hbm, dp_hbm, h_hbm, w0_hbm, b_hbm,
               outp_hbm, alpha_hbm,
               accf_v, den_v, beta_v, dpidx_b, dpsl_b,
               scan_b, posc_b, dlocc_b,
               pos_b, srcg_b, exg_b, eag_b, alphab_b,
               rows_v, w0_v, b_v):
    wid = lax.axis_index("s") * NC + lax.axis_index("c")
    base_node = wid * NT
    iota16 = lax.iota(jnp.int32, 16)
    zeros16 = jnp.zeros((16,), jnp.float32)

    def zacc(i, _):
        for u in range(4):
            accf_v[pl.ds(i * 64 + u * 16, 16)] = zeros16
        return 0
    lax.fori_loop(0, ACC_ROWS * D // 64, zacc, 0)

    def zsmall(i, _):
        den_v[pl.ds(i * 16, 16)] = zeros16
        beta_v[pl.ds(i * 16, 16)] = zeros16
        return 0
    lax.fori_loop(0, (NT + 16) // 16, zsmall, 0)

    def zidx(i, _):
        dpidx_b[pl.ds(i * 16, 16)] = base_node + i * 16 + iota16
        return 0
    lax.fori_loop(0, NT // 16, zidx, 0)

    pltpu.sync_copy(w0_hbm, w0_v)
    pltpu.sync_copy(b_hbm, b_v)

    for p in range(NW):
        pltpu.sync_copy(dp_hbm.at[dpidx_b], dpsl_b)

        def dred(i, _):
            sl = pl.ds(i * 16, 16)
            den_v[sl] = den_v[sl] + dpsl_b[sl]
            if p < NW - 1:
                dpidx_b[sl] = dpidx_b[sl] + NPAD
            return 0
        lax.fori_loop(0, NT // 16, dred, 0)

    def chunk(ch, cnt):
        ebch = ch * SCH
        pltpu.sync_copy(dst_hbm.at[pl.ds(ebch, SCH)], scan_b)

        def scan16(jv, cnt):
            sl = pl.ds(jv * 16, 16)
            dv = scan_b[sl]
            eidv = ebch + jv * 16 + iota16
            m = (dv >= base_node) & (dv < base_node + NT) & (eidv < E)
            coff = jnp.minimum(cnt, OWN_MAX)
            plsc.store_compressed(posc_b.at[pl.ds(coff, 16)], eidv, mask=m)
            plsc.store_compressed(
                dlocc_b.at[pl.ds(coff, 16)], dv - base_node, mask=m)
            return cnt + jnp.sum(m.astype(jnp.int32))
        return lax.fori_loop(0, SCH // 16, scan16, cnt)
    cnt = lax.fori_loop(0, NCH, chunk, jnp.int32(0))
    cnt = jnp.minimum(cnt, OWN_MAX)

    def pad_tail(i, _):
        posc_b[pl.ds(cnt + i * 16, 16)] = jnp.full((16,), E, jnp.int32)
        dlocc_b[pl.ds(cnt + i * 16, 16)] = jnp.full((16,), DUMPLOC, jnp.int32)
        return 0
    lax.fori_loop(0, RB // 16, pad_tail, 0)

    nbt = (cnt + RB - 1) // RB

    def batch(bt, _):
        off = bt * RB
        for k in range(RB // 16):
            pos_b[pl.ds(k * 16, 16)] = posc_b[pl.ds(off + k * 16, 16)]
        pltpu.sync_copy(src_hbm.at[pos_b], srcg_b)
        pltpu.sync_copy(ex_hbm.at[pos_b], exg_b)
        pltpu.sync_copy(ea_hbm.at[pos_b], eag_b)
        pltpu.sync_copy(h_hbm.at[srcg_b], rows_v)
        for k in range(RB // 16):
            sl = pl.ds(k * 16, 16)
            dlocv = dlocc_b[pl.ds(off + k * 16, 16)]
            denv = plsc.load_gather(den_v, [dlocv])
            alpha = exg_b[sl] / (denv + 1e-16)
            vmask = off + k * 16 + iota16 < cnt
            alpha = jnp.where(vmask, alpha, 0.0)
            alphab_b[sl] = alpha
            plsc.addupdate_scatter(
                beta_v, [dlocv], alpha * eag_b[sl], mask=vmask)
        pltpu.sync_copy(alphab_b, alpha_hbm.at[pos_b])

        def rowacc(r, _):
            r16 = jnp.broadcast_to(r, (16,)).astype(jnp.int32)
            dloc16 = plsc.load_gather(dlocc_b, [off + r16])
            a16 = plsc.load_gather(alphab_b, [r16])
            addr0 = dloc16 * D + iota16
            for q in range(D // 16):
                plsc.addupdate_scatter(
                    accf_v, [addr0 + q * 16],
                    rows_v[r, pl.ds(q * 16, 16)] * a16)
            return 0
        lax.fori_loop(0, RB, rowacc, 0)
        return 0
    lax.fori_loop(0, nbt, batch, 0)

    def fin(g, _):
        for j in range(16):
            row = g * 16 + j
            b16 = plsc.load_gather(
                beta_v, [jnp.broadcast_to(row, (16,)).astype(jnp.int32)])
            for q in range(D // 16):
                qs = pl.ds(q * 16, 16)
                rows_v[j, qs] = (accf_v[pl.ds(row * D + q * 16, 16)]
                                 + b16 * w0_v[qs] + b_v[qs])
        pltpu.sync_copy(rows_v.at[pl.ds(0, 16)],
                        outp_hbm.at[pl.ds(base_node + g * 16, 16)])
        return 0
    lax.fori_loop(0, NT // 16, fin, 0)


def kernel(x, edge_index, edge_attr, W, We, a_src, a_dst, a_edge, b):
    src = jnp.pad(edge_index[0], (0, E_PAD - E))
    dst = jnp.pad(edge_index[1], (0, E_PAD - E))
    ea = jnp.pad(edge_attr[:, 0], (0, E_PAD - E))
    asd8 = jnp.zeros((D, 8), jnp.float32).at[:, 0].set(a_src).at[:, 1].set(a_dst)
    ae2 = a_edge.reshape(1, D)

    h, sd = _project(x, W, asd8, We, ae2)
    s = sd[:, 0]
    d = sd[:, 1]
    cvec = sd[0:16, 2]

    ex, dparts = _edge_logits(src, dst, ea, s, d, cvec)
    outp, alpha = _aggregate(src, dst, ea, ex, dparts, h, We[0], b)

    return outp[:N], alpha[:E]


# speedup vs baseline: 161.2163x; 1.0968x over previous
import functools

import jax
import jax.numpy as jnp
from jax import lax
from jax.experimental import pallas as pl
from jax.experimental.pallas import tpu as pltpu
from jax.experimental.pallas import tpu_sc as plsc

N = 10000
E = 160000
D_IN = 258
D = 256

NC = 2
NS = 16
NW = NC * NS

EPW = 5120
E_PAD = NW * EPW
NPAD = 10240
NT = NPAD // NW
DUMPLOC = NT
ACC_ROWS = NT + 1
SCH = 512
NCH = E_PAD // SCH
RB = 64
OWN_MAX = 5632


def _mm_body(x_ref, w_ref, asd_ref, we_ref, ae_ref, h_ref, sd_ref):
    h = jnp.dot(x_ref[...], w_ref[...], preferred_element_type=jnp.float32)
    h_ref[...] = h
    sd = jnp.dot(h, asd_ref[...], preferred_element_type=jnp.float32)
    c = jnp.sum(we_ref[...] * ae_ref[...])
    col = lax.broadcasted_iota(jnp.int32, sd.shape, 1)
    sd_ref[...] = sd + jnp.where(col == 2, c, 0.0)


def _project(x, W, asd8, We, ae2):
    bm = 1000
    return pl.pallas_call(
        _mm_body,
        grid=(N // bm,),
        in_specs=[
            pl.BlockSpec((bm, D_IN), lambda i: (i, 0)),
            pl.BlockSpec((D_IN, D), lambda i: (0, 0)),
            pl.BlockSpec((D, 8), lambda i: (0, 0)),
            pl.BlockSpec((1, D), lambda i: (0, 0)),
            pl.BlockSpec((1, D), lambda i: (0, 0)),
        ],
        out_specs=[
            pl.BlockSpec((bm, D), lambda i: (i, 0)),
            pl.BlockSpec((bm, 8), lambda i: (i, 0)),
        ],
        out_shape=[
            jax.ShapeDtypeStruct((N, D), jnp.float32),
            jax.ShapeDtypeStruct((N, 8), jnp.float32),
        ],
    )(x, W, asd8, We, ae2)


_sc_mesh = plsc.VectorSubcoreMesh(core_axis_name="c", subcore_axis_name="s")


@functools.partial(
    pl.kernel,
    out_type=(
        jax.ShapeDtypeStruct((E_PAD,), jnp.float32),
        jax.ShapeDtypeStruct((NW * NPAD,), jnp.float32),
    ),
    mesh=_sc_mesh,
    compiler_params=pltpu.CompilerParams(needs_layout_passes=False),
    scratch_types=[
        pltpu.VMEM((N,), jnp.float32),
        pltpu.VMEM((N,), jnp.float32),
        pltpu.VMEM((16,), jnp.float32),
        pltpu.VMEM((EPW,), jnp.int32),
        pltpu.VMEM((EPW,), jnp.int32),
        pltpu.VMEM((EPW,), jnp.float32),
        pltpu.VMEM((EPW,), jnp.float32),
        pltpu.VMEM((NPAD,), jnp.float32),
    ],
)
def _edge_logits(src_hbm, dst_hbm, ea_hbm, s_hbm, d_hbm, c_hbm,
                 ex_hbm, dp_hbm,
                 s_v, d_v, c_v, src_v, dst_v, ea_v, ex_v, den_v):
    wid = lax.axis_index("s") * NC + lax.axis_index("c")
    base = wid * EPW

    def zero(i, _):
        den_v[pl.ds(i * 16, 16)] = jnp.zeros((16,), jnp.float32)
        return 0
    lax.fori_loop(0, NPAD // 16, zero, 0)

    pltpu.sync_copy(s_hbm, s_v)
    pltpu.sync_copy(d_hbm, d_v)
    pltpu.sync_copy(c_hbm, c_v)
    pltpu.sync_copy(src_hbm.at[pl.ds(base, EPW)], src_v)
    pltpu.sync_copy(dst_hbm.at[pl.ds(base, EPW)], dst_v)
    pltpu.sync_copy(ea_hbm.at[pl.ds(base, EPW)], ea_v)
    cc = c_v[...]

    def body(i, _):
        sl = pl.ds(i * 16, 16)
        sv = src_v[sl]
        dv = dst_v[sl]
        sg = plsc.load_gather(s_v, [sv])
        dg = plsc.load_gather(d_v, [jnp.minimum(dv, N - 1)])
        logit = sg + dg + cc * ea_v[sl]
        logit = jnp.maximum(logit, 0.2 * logit)
        ex = jnp.exp(logit)
        ex_v[sl] = ex
        eids = base + i * 16 + lax.iota(jnp.int32, 16)
        plsc.addupdate_scatter(den_v, [dv], ex, mask=eids < E)
        return 0
    lax.fori_loop(0, EPW // 16, body, 0)

    pltpu.sync_copy(ex_v, ex_hbm.at[pl.ds(base, EPW)])
    pltpu.sync_copy(den_v, dp_hbm.at[pl.ds(wid * NPAD, NPAD)])


@functools.partial(
    pl.kernel,
    out_type=(
        jax.ShapeDtypeStruct((NPAD, D), jnp.float32),
        jax.ShapeDtypeStruct((E_PAD,), jnp.float32),
    ),
    mesh=_sc_mesh,
    compiler_params=pltpu.CompilerParams(needs_layout_passes=False),
    scratch_types=[
        pltpu.VMEM((ACC_ROWS * D,), jnp.float32),
        pltpu.VMEM((NT + 8,), jnp.float32),
        pltpu.VMEM((NT + 8,), jnp.float32),
        pltpu.VMEM((NT,), jnp.int32),
        pltpu.VMEM((NT,), jnp.float32),
        pltpu.VMEM((SCH,), jnp.int32),
        pltpu.VMEM((SCH,), jnp.int32),
        pltpu.VMEM((OWN_MAX + 2 * RB + 16,), jnp.int32),
        pltpu.VMEM((OWN_MAX + 2 * RB + 16,), jnp.int32),
        pltpu.VMEM((RB,), jnp.int32),
        pltpu.VMEM((RB,), jnp.int32),
        pltpu.VMEM((RB,), jnp.int32),
        pltpu.VMEM((RB,), jnp.int32),
        pltpu.VMEM((RB,), jnp.float32),
        pltpu.VMEM((RB,), jnp.float32),
        pltpu.VMEM((RB,), jnp.float32),
        pltpu.VMEM((RB,), jnp.float32),
        pltpu.VMEM((RB,), jnp.float32),
        pltpu.VMEM((RB,), jnp.float32),
        pltpu.VMEM((RB, D), jnp.float32),
        pltpu.VMEM((RB, D), jnp.float32),
        pltpu.VMEM((D,), jnp.float32),
        pltpu.VMEM((D,), jnp.float32),
        pltpu.SemaphoreType.DMA,
        pltpu.SemaphoreType.DMA,
        pltpu.SemaphoreType.DMA,
        pltpu.SemaphoreType.DMA,
        pltpu.SemaphoreType.DMA,
        pltpu.SemaphoreType.DMA,
        pltpu.SemaphoreType.DMA,
        pltpu.SemaphoreType.DMA,
    ],
)
def _aggregate(src_hbm, dst_hbm, ea_hbm, ex_hbm, dp_hbm, h_hbm, w0_hbm, b_hbm,
               outp_hbm, alpha_hbm,
               accf_v, den_v, beta_v, dpidx_b, dpsl_b,
               scana_b, scanb_b, posc_b, dlocc_b,
               posa_b, posb_b, srcga_b, srcgb_b, exga_b, exgb_b,
               eaga_b, eagb_b, alphaba_b, alphabb_b, rowsa_v, rowsb_v,
               w0_v, b_v,
               sem_a, sem_b, hsa0, hsa1, hsb0, hsb1, asema, asemb):
    wid = lax.axis_index("s") * NC + lax.axis_index("c")
    base_node = wid * NT
    iota16 = lax.iota(jnp.int32, 16)
    zeros16 = jnp.zeros((16,), jnp.float32)

    def zacc(i, _):
        for u in range(4):
            accf_v[pl.ds(i * 64 + u * 16, 16)] = zeros16
        return 0
    lax.fori_loop(0, ACC_ROWS * D // 64, zacc, 0)

    def zsmall(i, _):
        den_v[pl.ds(i * 16, 16)] = zeros16
        beta_v[pl.ds(i * 16, 16)] = zeros16
        return 0
    lax.fori_loop(0, (NT + 8) // 16, zsmall, 0)

    def zidx(i, _):
        dpidx_b[pl.ds(i * 16, 16)] = base_node + i * 16 + iota16
        return 0
    lax.fori_loop(0, NT // 16, zidx, 0)

    for p in range(NW):
        pltpu.sync_copy(dp_hbm.at[dpidx_b], dpsl_b)

        def dred(i, _):
            sl = pl.ds(i * 16, 16)
            den_v[sl] = den_v[sl] + dpsl_b[sl]
            if p < NW - 1:
                dpidx_b[sl] = dpidx_b[sl] + NPAD
            return 0
        lax.fori_loop(0, NT // 16, dred, 0)

    def scan_issue(ch, buf, sem):
        ebch = jnp.minimum(ch, NCH - 1) * SCH
        pltpu.async_copy(dst_hbm.at[pl.ds(ebch, SCH)], buf, sem)

    def scan_wait(buf, sem):
        pltpu.make_async_copy(dst_hbm.at[pl.ds(0, SCH)], buf, sem).wait()

    def compact(ch, buf, cnt):
        ebch = ch * SCH

        def scan16(jv, cnt):
            sl = pl.ds(jv * 16, 16)
            dv = buf[sl]
            eidv = ebch + jv * 16 + iota16
            dloc = dv - base_node
            m = (dloc >= 0) & (dloc < NT)
            coff = jnp.minimum(cnt, OWN_MAX)
            plsc.store_compressed(posc_b.at[pl.ds(coff, 16)], eidv, mask=m)
            plsc.store_compressed(dlocc_b.at[pl.ds(coff, 16)], dloc, mask=m)
            return cnt + jnp.sum(m.astype(jnp.int32))
        return lax.fori_loop(0, SCH // 16, scan16, cnt)

    with jax.named_scope("k2_scan"):
        scan_issue(0, scana_b, sem_a)
        scan_issue(1, scanb_b, sem_b)

        def chunkpair(p2, cnt):
            ch = 2 * p2
            scan_wait(scana_b, sem_a)
            cnt = compact(ch, scana_b, cnt)
            scan_issue(ch + 2, scana_b, sem_a)
            scan_wait(scanb_b, sem_b)
            cnt = compact(ch + 1, scanb_b, cnt)
            scan_issue(ch + 3, scanb_b, sem_b)
            return cnt
        cnt = lax.fori_loop(0, NCH // 2, chunkpair, jnp.int32(0))
        scan_wait(scana_b, sem_a)
        scan_wait(scanb_b, sem_b)
    cnt = jnp.minimum(cnt, OWN_MAX)

    def pad_tail(i, _):
        posc_b[pl.ds(cnt + i * 16, 16)] = jnp.full((16,), E, jnp.int32)
        dlocc_b[pl.ds(cnt + i * 16, 16)] = jnp.full((16,), DUMPLOC, jnp.int32)
        return 0
    lax.fori_loop(0, 2 * RB // 16, pad_tail, 0)

    nbt = (cnt + RB - 1) // RB
    npair = (nbt + 1) // 2

    def prep(bt, pos_b, srcg_b, exg_b, eag_b, sem):
        off = bt * RB
        for k in range(RB // 16):
            pos_b[pl.ds(k * 16, 16)] = posc_b[pl.ds(off + k * 16, 16)]
        pltpu.async_copy(src_hbm.at[pos_b], srcg_b, sem)
        pltpu.async_copy(ex_hbm.at[pos_b], exg_b, sem)
        pltpu.async_copy(ea_hbm.at[pos_b], eag_b, sem)

    def mid(pos_b, srcg_b, exg_b, eag_b, sem, rows_v, hs0, hs1):
        pltpu.make_async_copy(src_hbm.at[pos_b], srcg_b, sem).wait()
        pltpu.make_async_copy(ex_hbm.at[pos_b], exg_b, sem).wait()
        pltpu.make_async_copy(ea_hbm.at[pos_b], eag_b, sem).wait()
        hh = RB // 2
        pltpu.async_copy(h_hbm.at[srcg_b.at[pl.ds(0, hh)]],
                         rows_v.at[pl.ds(0, hh)], hs0)
        pltpu.async_copy(h_hbm.at[srcg_b.at[pl.ds(hh, hh)]],
                         rows_v.at[pl.ds(hh, hh)], hs1)

    def finb(bt, pos_b, srcg_b, exg_b, eag_b, alphab_b, rows_v,
             hs0, hs1, asem):
        off = bt * RB
        for k in range(RB // 16):
            sl = pl.ds(k * 16, 16)
            dlocv = dlocc_b[pl.ds(off + k * 16, 16)]
            denv = plsc.load_gather(den_v, [dlocv])
            alpha = exg_b[sl] / (denv + 1e-16)
            vmask = off + k * 16 + iota16 < cnt
            alpha = jnp.where(vmask, alpha, 0.0)
            alphab_b[sl] = alpha
            plsc.addupdate_scatter(
                beta_v, [dlocv], alpha * eag_b[sl], mask=vmask)
        pltpu.async_copy(alphab_b, alpha_hbm.at[pos_b], asem)
        hh = RB // 2
        pltpu.make_async_copy(h_hbm.at[srcg_b.at[pl.ds(0, hh)]],
                              rows_v.at[pl.ds(0, hh)], hs0).wait()
        pltpu.make_async_copy(h_hbm.at[srcg_b.at[pl.ds(hh, hh)]],
                              rows_v.at[pl.ds(hh, hh)], hs1).wait()

        def rowacc(r4, _):
            for u in range(4):
                r = r4 * 4 + u
                r16 = jnp.broadcast_to(r, (16,)).astype(jnp.int32)
                dloc16 = plsc.load_gather(dlocc_b, [off + r16])
                a16 = plsc.load_gather(alphab_b, [r16])
                addr0 = dloc16 * D + iota16
                for q in range(D // 16):
                    plsc.addupdate_scatter(
                        accf_v, [addr0 + q * 16],
                        rows_v[r, pl.ds(q * 16, 16)] * a16)
            return 0
        lax.fori_loop(0, RB // 4, rowacc, 0)

    def adrain(alphab_b, pos_b, asem):
        pltpu.make_async_copy(alphab_b, alpha_hbm.at[pos_b], asem).wait()

    with jax.named_scope("k2_batches"):
        prep(0, posa_b, srcga_b, exga_b, eaga_b, sem_a)
        prep(1, posb_b, srcgb_b, exgb_b, eagb_b, sem_b)

        def pair(p2, _):
            bt = 2 * p2
            mid(posa_b, srcga_b, exga_b, eaga_b, sem_a, rowsa_v, hsa0, hsa1)
            mid(posb_b, srcgb_b, exgb_b, eagb_b, sem_b, rowsb_v, hsb0, hsb1)
            finb(bt, posa_b, srcga_b, exga_b, eaga_b, alphaba_b, rowsa_v,
                 hsa0, hsa1, asema)
            adrain(alphaba_b, posa_b, asema)
            prep(bt + 2, posa_b, srcga_b, exga_b, eaga_b, sem_a)
            finb(bt + 1, posb_b, srcgb_b, exgb_b, eagb_b, alphabb_b, rowsb_v,
                 hsb0, hsb1, asemb)
            adrain(alphabb_b, posb_b, asemb)
            prep(bt + 3, posb_b, srcgb_b, exgb_b, eagb_b, sem_b)
            return 0
        lax.fori_loop(0, npair, pair, 0)
        pltpu.make_async_copy(src_hbm.at[posa_b], srcga_b, sem_a).wait()
        pltpu.make_async_copy(ex_hbm.at[posa_b], exga_b, sem_a).wait()
        pltpu.make_async_copy(ea_hbm.at[posa_b], eaga_b, sem_a).wait()
        pltpu.make_async_copy(src_hbm.at[posb_b], srcgb_b, sem_b).wait()
        pltpu.make_async_copy(ex_hbm.at[posb_b], exgb_b, sem_b).wait()
        pltpu.make_async_copy(ea_hbm.at[posb_b], eagb_b, sem_b).wait()

    pltpu.sync_copy(w0_hbm, w0_v)
    pltpu.sync_copy(b_hbm, b_v)

    def fin(g, _):
        for j in range(16):
            row = g * 16 + j
            b16 = plsc.load_gather(
                beta_v, [jnp.broadcast_to(row, (16,)).astype(jnp.int32)])
            for q in range(D // 16):
                qs = pl.ds(q * 16, 16)
                rowsa_v[j, qs] = (accf_v[pl.ds(row * D + q * 16, 16)]
                                  + b16 * w0_v[qs] + b_v[qs])
        pltpu.sync_copy(rowsa_v.at[pl.ds(0, 16)],
                        outp_hbm.at[pl.ds(base_node + g * 16, 16)])
        return 0
    with jax.named_scope("k2_fin"):
        lax.fori_loop(0, NT // 16, fin, 0)


def kernel(x, edge_index, edge_attr, W, We, a_src, a_dst, a_edge, b):
    src = jnp.pad(edge_index[0], (0, E_PAD - E))
    dst = jnp.pad(edge_index[1], (0, E_PAD - E), constant_values=NPAD - 1)
    ea = jnp.pad(edge_attr[:, 0], (0, E_PAD - E))
    asd8 = jnp.zeros((D, 8), jnp.float32).at[:, 0].set(a_src).at[:, 1].set(a_dst)
    ae2 = a_edge.reshape(1, D)

    h, sd = _project(x, W, asd8, We, ae2)
    s = sd[:, 0]
    d = sd[:, 1]
    cvec = sd[0:16, 2]

    ex, dparts = _edge_logits(src, dst, ea, s, d, cvec)
    outp, alpha = _aggregate(src, dst, ea, ex, dparts, h, We[0], b)

    return outp[:N], alpha[:E]


# speedup vs baseline: 162.0686x; 1.0053x over previous
import functools

import jax
import jax.numpy as jnp
from jax import lax
from jax.experimental import pallas as pl
from jax.experimental.pallas import tpu as pltpu
from jax.experimental.pallas import tpu_sc as plsc

N = 10000
E = 160000
D_IN = 258
D = 256

NC = 2
NS = 16
NW = NC * NS

EPW = 5120
E_PAD = NW * EPW
NPAD = 10240
NT = NPAD // NW
DUMPLOC = NT
ACC_ROWS = NT + 1
SCH = 512
NCH = E_PAD // SCH
RB = 64
OWN_MAX = 5632


def _mm_body(x_ref, w_ref, asd_ref, we_ref, ae_ref, h_ref, sd_ref):
    h = jnp.dot(x_ref[...], w_ref[...], preferred_element_type=jnp.float32)
    h_ref[...] = h
    sd = jnp.dot(h, asd_ref[...], preferred_element_type=jnp.float32)
    c = jnp.sum(we_ref[...] * ae_ref[...])
    col = lax.broadcasted_iota(jnp.int32, sd.shape, 1)
    sd_ref[...] = sd + jnp.where(col == 2, c, 0.0)


def _project(x, W, asd8, We, ae2):
    bm = 1000
    return pl.pallas_call(
        _mm_body,
        grid=(N // bm,),
        in_specs=[
            pl.BlockSpec((bm, D_IN), lambda i: (i, 0)),
            pl.BlockSpec((D_IN, D), lambda i: (0, 0)),
            pl.BlockSpec((D, 8), lambda i: (0, 0)),
            pl.BlockSpec((1, D), lambda i: (0, 0)),
            pl.BlockSpec((1, D), lambda i: (0, 0)),
        ],
        out_specs=[
            pl.BlockSpec((bm, D), lambda i: (i, 0)),
            pl.BlockSpec((bm, 8), lambda i: (i, 0)),
        ],
        out_shape=[
            jax.ShapeDtypeStruct((N, D), jnp.float32),
            jax.ShapeDtypeStruct((N, 8), jnp.float32),
        ],
    )(x, W, asd8, We, ae2)


_sc_mesh = plsc.VectorSubcoreMesh(core_axis_name="c", subcore_axis_name="s")


@functools.partial(
    pl.kernel,
    out_type=(
        jax.ShapeDtypeStruct((E_PAD,), jnp.float32),
        jax.ShapeDtypeStruct((NW * NPAD,), jnp.float32),
    ),
    mesh=_sc_mesh,
    compiler_params=pltpu.CompilerParams(needs_layout_passes=False),
    scratch_types=[
        pltpu.VMEM((N,), jnp.float32),
        pltpu.VMEM((N,), jnp.float32),
        pltpu.VMEM((16,), jnp.float32),
        pltpu.VMEM((EPW,), jnp.int32),
        pltpu.VMEM((EPW,), jnp.int32),
        pltpu.VMEM((EPW,), jnp.float32),
        pltpu.VMEM((EPW,), jnp.float32),
        pltpu.VMEM((NPAD,), jnp.float32),
    ],
)
def _edge_logits(src_hbm, dst_hbm, ea_hbm, s_hbm, d_hbm, c_hbm,
                 ex_hbm, dp_hbm,
                 s_v, d_v, c_v, src_v, dst_v, ea_v, ex_v, den_v):
    wid = lax.axis_index("s") * NC + lax.axis_index("c")
    base = wid * EPW

    def zero(i, _):
        den_v[pl.ds(i * 16, 16)] = jnp.zeros((16,), jnp.float32)
        return 0
    lax.fori_loop(0, NPAD // 16, zero, 0)

    pltpu.sync_copy(s_hbm, s_v)
    pltpu.sync_copy(d_hbm, d_v)
    pltpu.sync_copy(c_hbm, c_v)
    pltpu.sync_copy(src_hbm.at[pl.ds(base, EPW)], src_v)
    pltpu.sync_copy(dst_hbm.at[pl.ds(base, EPW)], dst_v)
    pltpu.sync_copy(ea_hbm.at[pl.ds(base, EPW)], ea_v)
    cc = c_v[...]

    def body(i, _):
        sl = pl.ds(i * 16, 16)
        sv = src_v[sl]
        dv = dst_v[sl]
        sg = plsc.load_gather(s_v, [sv])
        dg = plsc.load_gather(d_v, [jnp.minimum(dv, N - 1)])
        logit = sg + dg + cc * ea_v[sl]
        logit = jnp.maximum(logit, 0.2 * logit)
        ex = jnp.exp(logit)
        ex_v[sl] = ex
        eids = base + i * 16 + lax.iota(jnp.int32, 16)
        plsc.addupdate_scatter(den_v, [dv], ex, mask=eids < E)
        return 0
    lax.fori_loop(0, EPW // 16, body, 0)

    pltpu.sync_copy(ex_v, ex_hbm.at[pl.ds(base, EPW)])
    pltpu.sync_copy(den_v, dp_hbm.at[pl.ds(wid * NPAD, NPAD)])


@functools.partial(
    pl.kernel,
    out_type=(
        jax.ShapeDtypeStruct((NPAD, D), jnp.float32),
        jax.ShapeDtypeStruct((E_PAD,), jnp.float32),
    ),
    mesh=_sc_mesh,
    compiler_params=pltpu.CompilerParams(needs_layout_passes=False),
    scratch_types=[
        pltpu.VMEM((ACC_ROWS * D,), jnp.float32),
        pltpu.VMEM((NT + 8,), jnp.float32),
        pltpu.VMEM((NT + 8,), jnp.float32),
        pltpu.VMEM((NT,), jnp.int32),
        pltpu.VMEM((NT,), jnp.float32),
        pltpu.VMEM((SCH,), jnp.int32),
        pltpu.VMEM((SCH,), jnp.int32),
        pltpu.VMEM((OWN_MAX + 2 * RB + 16,), jnp.int32),
        pltpu.VMEM((OWN_MAX + 2 * RB + 16,), jnp.int32),
        pltpu.VMEM((RB,), jnp.int32),
        pltpu.VMEM((RB,), jnp.int32),
        pltpu.VMEM((RB,), jnp.int32),
        pltpu.VMEM((RB,), jnp.int32),
        pltpu.VMEM((RB,), jnp.float32),
        pltpu.VMEM((RB,), jnp.float32),
        pltpu.VMEM((RB,), jnp.float32),
        pltpu.VMEM((RB,), jnp.float32),
        pltpu.VMEM((RB,), jnp.float32),
        pltpu.VMEM((RB,), jnp.float32),
        pltpu.VMEM((RB, D), jnp.float32),
        pltpu.VMEM((RB, D), jnp.float32),
        pltpu.VMEM((D,), jnp.float32),
        pltpu.VMEM((D,), jnp.float32),
        pltpu.SemaphoreType.DMA,
        pltpu.SemaphoreType.DMA,
        pltpu.SemaphoreType.DMA,
        pltpu.SemaphoreType.DMA,
        pltpu.SemaphoreType.DMA,
        pltpu.SemaphoreType.DMA,
        pltpu.SemaphoreType.DMA,
        pltpu.SemaphoreType.DMA,
    ],
)
def _aggregate(src_hbm, dst_hbm, ea_hbm, ex_hbm, dp_hbm, h_hbm, w0_hbm, b_hbm,
               outp_hbm, alpha_hbm,
               accf_v, den_v, beta_v, dpidx_b, dpsl_b,
               scana_b, scanb_b, posc_b, dlocc_b,
               posa_b, posb_b, srcga_b, srcgb_b, exga_b, exgb_b,
               eaga_b, eagb_b, alphaba_b, alphabb_b, rowsa_v, rowsb_v,
               w0_v, b_v,
               sem_a, sem_b, hsa0, hsa1, hsb0, hsb1, asema, asemb):
    wid = lax.axis_index("s") * NC + lax.axis_index("c")
    base_node = wid * NT
    iota16 = lax.iota(jnp.int32, 16)
    zeros16 = jnp.zeros((16,), jnp.float32)

    def zacc(i, _):
        for u in range(4):
            accf_v[pl.ds(i * 64 + u * 16, 16)] = zeros16
        return 0
    lax.fori_loop(0, ACC_ROWS * D // 64, zacc, 0)

    def zsmall(i, _):
        den_v[pl.ds(i * 16, 16)] = zeros16
        beta_v[pl.ds(i * 16, 16)] = zeros16
        return 0
    lax.fori_loop(0, (NT + 8) // 16, zsmall, 0)

    def zidx(i, _):
        dpidx_b[pl.ds(i * 16, 16)] = base_node + i * 16 + iota16
        return 0
    lax.fori_loop(0, NT // 16, zidx, 0)

    for p in range(NW):
        pltpu.sync_copy(dp_hbm.at[dpidx_b], dpsl_b)

        def dred(i, _):
            sl = pl.ds(i * 16, 16)
            den_v[sl] = den_v[sl] + dpsl_b[sl]
            if p < NW - 1:
                dpidx_b[sl] = dpidx_b[sl] + NPAD
            return 0
        lax.fori_loop(0, NT // 16, dred, 0)

    def scan_issue(ch, buf, sem):
        ebch = jnp.minimum(ch, NCH - 1) * SCH
        pltpu.async_copy(dst_hbm.at[pl.ds(ebch, SCH)], buf, sem)

    def scan_wait(buf, sem):
        pltpu.make_async_copy(dst_hbm.at[pl.ds(0, SCH)], buf, sem).wait()

    def compact(ch, buf, cnt):
        ebch = ch * SCH

        def scan16(jv, cnt):
            sl = pl.ds(jv * 16, 16)
            dv = buf[sl]
            eidv = ebch + jv * 16 + iota16
            dloc = dv - base_node
            m = (dloc >= 0) & (dloc < NT)
            coff = jnp.minimum(cnt, OWN_MAX)
            plsc.store_compressed(posc_b.at[pl.ds(coff, 16)], eidv, mask=m)
            plsc.store_compressed(dlocc_b.at[pl.ds(coff, 16)], dloc, mask=m)
            return cnt + plsc.all_reduce_population_count(m)[0]
        return lax.fori_loop(0, SCH // 16, scan16, cnt)

    with jax.named_scope("k2_scan"):
        scan_issue(0, scana_b, sem_a)
        scan_issue(1, scanb_b, sem_b)

        def chunkpair(p2, cnt):
            ch = 2 * p2
            scan_wait(scana_b, sem_a)
            cnt = compact(ch, scana_b, cnt)
            scan_issue(ch + 2, scana_b, sem_a)
            scan_wait(scanb_b, sem_b)
            cnt = compact(ch + 1, scanb_b, cnt)
            scan_issue(ch + 3, scanb_b, sem_b)
            return cnt
        cnt = lax.fori_loop(0, NCH // 2, chunkpair, jnp.int32(0))
        scan_wait(scana_b, sem_a)
        scan_wait(scanb_b, sem_b)
    cnt = jnp.minimum(cnt, OWN_MAX)

    def pad_tail(i, _):
        posc_b[pl.ds(cnt + i * 16, 16)] = jnp.full((16,), E, jnp.int32)
        dlocc_b[pl.ds(cnt + i * 16, 16)] = jnp.full((16,), DUMPLOC, jnp.int32)
        return 0
    lax.fori_loop(0, 2 * RB // 16, pad_tail, 0)

    nbt = (cnt + RB - 1) // RB
    npair = (nbt + 1) // 2

    def prep(bt, pos_b, srcg_b, exg_b, eag_b, sem):
        off = bt * RB
        for k in range(RB // 16):
            pos_b[pl.ds(k * 16, 16)] = posc_b[pl.ds(off + k * 16, 16)]
        pltpu.async_copy(src_hbm.at[pos_b], srcg_b, sem)
        pltpu.async_copy(ex_hbm.at[pos_b], exg_b, sem)
        pltpu.async_copy(ea_hbm.at[pos_b], eag_b, sem)

    def mid(pos_b, srcg_b, exg_b, eag_b, sem, rows_v, hs0, hs1):
        pltpu.make_async_copy(src_hbm.at[pos_b], srcg_b, sem).wait()
        pltpu.make_async_copy(ex_hbm.at[pos_b], exg_b, sem).wait()
        pltpu.make_async_copy(ea_hbm.at[pos_b], eag_b, sem).wait()
        hh = RB // 2
        pltpu.async_copy(h_hbm.at[srcg_b.at[pl.ds(0, hh)]],
                         rows_v.at[pl.ds(0, hh)], hs0)
        pltpu.async_copy(h_hbm.at[srcg_b.at[pl.ds(hh, hh)]],
                         rows_v.at[pl.ds(hh, hh)], hs1)

    def finb(bt, pos_b, srcg_b, exg_b, eag_b, alphab_b, rows_v,
             hs0, hs1, asem):
        off = bt * RB
        for k in range(RB // 16):
            sl = pl.ds(k * 16, 16)
            dlocv = dlocc_b[pl.ds(off + k * 16, 16)]
            denv = plsc.load_gather(den_v, [dlocv])
            alpha = exg_b[sl] / (denv + 1e-16)
            vmask = off + k * 16 + iota16 < cnt
            alpha = jnp.where(vmask, alpha, 0.0)
            alphab_b[sl] = alpha
            plsc.addupdate_scatter(
                beta_v, [dlocv], alpha * eag_b[sl], mask=vmask)
        pltpu.async_copy(alphab_b, alpha_hbm.at[pos_b], asem)
        hh = RB // 2
        pltpu.make_async_copy(h_hbm.at[srcg_b.at[pl.ds(0, hh)]],
                              rows_v.at[pl.ds(0, hh)], hs0).wait()
        pltpu.make_async_copy(h_hbm.at[srcg_b.at[pl.ds(hh, hh)]],
                              rows_v.at[pl.ds(hh, hh)], hs1).wait()

        def rowacc(r4, _):
            for u in range(4):
                r = r4 * 4 + u
                r16 = jnp.broadcast_to(r, (16,)).astype(jnp.int32)
                dloc16 = plsc.load_gather(dlocc_b, [off + r16])
                a16 = plsc.load_gather(alphab_b, [r16])
                addr0 = dloc16 * D + iota16
                for q in range(D // 16):
                    plsc.addupdate_scatter(
                        accf_v, [addr0 + q * 16],
                        rows_v[r, pl.ds(q * 16, 16)] * a16)
            return 0
        lax.fori_loop(0, RB // 4, rowacc, 0)

    def adrain(alphab_b, pos_b, asem):
        pltpu.make_async_copy(alphab_b, alpha_hbm.at[pos_b], asem).wait()

    with jax.named_scope("k2_batches"):
        prep(0, posa_b, srcga_b, exga_b, eaga_b, sem_a)
        prep(1, posb_b, srcgb_b, exgb_b, eagb_b, sem_b)

        def pair(p2, _):
            bt = 2 * p2
            mid(posa_b, srcga_b, exga_b, eaga_b, sem_a, rowsa_v, hsa0, hsa1)
            mid(posb_b, srcgb_b, exgb_b, eagb_b, sem_b, rowsb_v, hsb0, hsb1)
            finb(bt, posa_b, srcga_b, exga_b, eaga_b, alphaba_b, rowsa_v,
                 hsa0, hsa1, asema)
            adrain(alphaba_b, posa_b, asema)
            prep(bt + 2, posa_b, srcga_b, exga_b, eaga_b, sem_a)
            finb(bt + 1, posb_b, srcgb_b, exgb_b, eagb_b, alphabb_b, rowsb_v,
                 hsb0, hsb1, asemb)
            adrain(alphabb_b, posb_b, asemb)
            prep(bt + 3, posb_b, srcgb_b, exgb_b, eagb_b, sem_b)
            return 0
        lax.fori_loop(0, npair, pair, 0)
        pltpu.make_async_copy(src_hbm.at[posa_b], srcga_b, sem_a).wait()
        pltpu.make_async_copy(ex_hbm.at[posa_b], exga_b, sem_a).wait()
        pltpu.make_async_copy(ea_hbm.at[posa_b], eaga_b, sem_a).wait()
        pltpu.make_async_copy(src_hbm.at[posb_b], srcgb_b, sem_b).wait()
        pltpu.make_async_copy(ex_hbm.at[posb_b], exgb_b, sem_b).wait()
        pltpu.make_async_copy(ea_hbm.at[posb_b], eagb_b, sem_b).wait()

    pltpu.sync_copy(w0_hbm, w0_v)
    pltpu.sync_copy(b_hbm, b_v)

    def fin(g, _):
        for j in range(16):
            row = g * 16 + j
            b16 = plsc.load_gather(
                beta_v, [jnp.broadcast_to(row, (16,)).astype(jnp.int32)])
            for q in range(D // 16):
                qs = pl.ds(q * 16, 16)
                rowsa_v[j, qs] = (accf_v[pl.ds(row * D + q * 16, 16)]
                                  + b16 * w0_v[qs] + b_v[qs])
        pltpu.sync_copy(rowsa_v.at[pl.ds(0, 16)],
                        outp_hbm.at[pl.ds(base_node + g * 16, 16)])
        return 0
    with jax.named_scope("k2_fin"):
        lax.fori_loop(0, NT // 16, fin, 0)


def kernel(x, edge_index, edge_attr, W, We, a_src, a_dst, a_edge, b):
    src = jnp.pad(edge_index[0], (0, E_PAD - E))
    dst = jnp.pad(edge_index[1], (0, E_PAD - E), constant_values=NPAD - 1)
    ea = jnp.pad(edge_attr[:, 0], (0, E_PAD - E))
    asd8 = jnp.zeros((D, 8), jnp.float32).at[:, 0].set(a_src).at[:, 1].set(a_dst)
    ae2 = a_edge.reshape(1, D)

    h, sd = _project(x, W, asd8, We, ae2)
    s = sd[:, 0]
    d = sd[:, 1]
    cvec = sd[0:16, 2]

    ex, dparts = _edge_logits(src, dst, ea, s, d, cvec)
    outp, alpha = _aggregate(src, dst, ea, ex, dparts, h, We[0], b)

    return outp[:N], alpha[:E]
